# baseline (speedup 1.0000x reference)
import functools

import jax
import jax.numpy as jnp
from jax import lax
from jax.experimental import pallas as pl
from jax.experimental.pallas import tpu as pltpu
from jax.experimental.pallas import tpu_sc as plsc

_VC = 11008
_TAIL = 1024


def _sc_scan_gather(idx_flat, n_feat, tab_v, tab_tail):
    F = n_feat
    B = idx_flat.shape[0] // F
    K, V = tab_v.shape
    n_slabs = K // 8
    n_win = V // _VC
    tail_lo = n_win * _VC
    assert V - tail_lo <= _TAIL
    n_g = B // 16

    mesh = plsc.VectorSubcoreMesh(core_axis_name="c", subcore_axis_name="s")

    @functools.partial(
        pl.kernel,
        mesh=mesh,
        out_type=jax.ShapeDtypeStruct((K, B), jnp.float32),
        scratch_types=[
            pltpu.VMEM((B,), jnp.int32),
            pltpu.VMEM((8, _VC), jnp.float32),
            pltpu.VMEM((8, B), jnp.float32),
            pltpu.SemaphoreType.DMA,
        ],
        compiler_params=pltpu.CompilerParams(
            use_tc_tiling_on_sc=True, needs_layout_passes=False
        ),
    )
    def scan_gather(idx_hbm, tab_hbm, tail_hbm, out_hbm, idx_v, buf_v, out_v, sem_o):
        tid = lax.axis_index("s") * 2 + lax.axis_index("c")

        def gather_win(buf_base, cov_w):
            def _g(g, carry):
                acc = []
                for u in range(8):
                    col = g * 128 + u * 16
                    vv = idx_v[pl.ds(col, 16)]
                    lo = vv - buf_base
                    msk = plsc.bitcast(lo, jnp.uint32) < jnp.uint32(cov_w)
                    pos = lax.iota(jnp.int32, 16) + col
                    vals = [
                        plsc.load_gather(
                            buf_v, [jnp.full((16,), e, jnp.int32), lo], mask=msk
                        )
                        for e in range(8)
                    ]
                    acc.append((pos, msk, vals))
                for pos, msk, vals in acc:
                    for e in range(8):
                        plsc.store_scatter(
                            out_v,
                            [jnp.full((16,), e, jnp.int32), pos],
                            vals[e],
                            mask=msk,
                        )
                return carry

            lax.fori_loop(0, n_g // 8, _g, 0)

        def slab_work(jj, carry):
            slab = tid + 32 * jj

            @pl.when(slab < n_slabs)
            def _process():
                f = slab // 4
                r0 = slab * 8
                pltpu.sync_copy(idx_hbm.at[pl.ds(f * B, B)], idx_v)

                @pl.when(jj > 0)
                def _drain():
                    pltpu.make_async_copy(
                        out_v, out_hbm.at[pl.ds(r0, 8), :], sem_o
                    ).wait()
                for w in range(n_win):
                    pltpu.sync_copy(
                        tab_hbm.at[pl.ds(r0, 8), pl.ds(w * _VC, _VC)], buf_v
                    )
                    gather_win(w * _VC, _VC)
                pltpu.sync_copy(
                    tail_hbm.at[pl.ds(r0, 8), :], buf_v.at[:, pl.ds(0, _TAIL)]
                )
                gather_win(V - _TAIL, _TAIL)
                pltpu.async_copy(out_v, out_hbm.at[pl.ds(r0, 8), :], sem_o)

            return carry

        lax.fori_loop(0, (n_slabs + 31) // 32, slab_work, 0)
        pltpu.make_async_copy(out_v, out_hbm.at[pl.ds(0, 8), :], sem_o).wait()

    return scan_gather(idx_flat, tab_v, tab_tail)


def _tc_dense_softmax(g_t, W, b, block_b):
    K, B = g_t.shape
    out_dim = W.shape[1]

    def mm_kernel(g_ref, w_ref, b_ref, o_ref):
        logits = lax.dot_general(
            g_ref[...],
            w_ref[...],
            (((0,), (0,)), ((), ())),
            preferred_element_type=jnp.float32,
        )
        logits = logits + b_ref[...]
        m = jnp.max(logits, axis=-1, keepdims=True)
        e = jnp.exp(logits - m)
        o_ref[...] = e / jnp.sum(e, axis=-1, keepdims=True)

    return pl.pallas_call(
        mm_kernel,
        grid=(B // block_b,),
        in_specs=[
            pl.BlockSpec((K, block_b), lambda i: (0, i)),
            pl.BlockSpec((K, out_dim), lambda i: (0, 0)),
            pl.BlockSpec((1, out_dim), lambda i: (0, 0)),
        ],
        out_specs=pl.BlockSpec((block_b, out_dim), lambda i: (i, 0)),
        out_shape=jax.ShapeDtypeStruct((B, out_dim), jnp.float32),
    )(g_t, W, b.reshape(1, out_dim))


def kernel(inputs, batch_size, tables, W, b):
    F, V, E = tables.shape
    B = inputs.shape[0]
    tab_v = tables.transpose(0, 2, 1).reshape(F * E, V)
    idx_flat = inputs.T.reshape(F * B)
    tab_tail = lax.slice(tab_v, (0, V - _TAIL), (F * E, V))

    g_t = _sc_scan_gather(idx_flat, F, tab_v, tab_tail)
    return _tc_dense_softmax(g_t, W, b, block_b=512)


# speedup vs baseline: 1.2596x; 1.2596x over previous
import functools

import jax
import jax.numpy as jnp
from jax import lax
from jax.experimental import pallas as pl
from jax.experimental.pallas import tpu as pltpu
from jax.experimental.pallas import tpu_sc as plsc

_VC = 5504
_TAIL = 1024


def _sc_scan_gather(idx_flat, n_feat, tab_v, tab_tail):
    F = n_feat
    B = idx_flat.shape[0] // F
    K, V = tab_v.shape
    n_slabs = K // 8
    n_win = V // _VC
    tail_lo = n_win * _VC
    assert V - tail_lo <= _TAIL
    n_g = B // 16

    mesh = plsc.VectorSubcoreMesh(core_axis_name="c", subcore_axis_name="s")

    @functools.partial(
        pl.kernel,
        mesh=mesh,
        out_type=jax.ShapeDtypeStruct((K, B), jnp.float32),
        scratch_types=[
            pltpu.VMEM((B,), jnp.int32),
            pltpu.VMEM((8, _VC), jnp.float32),
            pltpu.VMEM((8, _VC), jnp.float32),
            pltpu.VMEM((8, B), jnp.float32),
            pltpu.SemaphoreType.DMA,
            pltpu.SemaphoreType.DMA,
            pltpu.SemaphoreType.DMA,
        ],
        compiler_params=pltpu.CompilerParams(
            use_tc_tiling_on_sc=True, needs_layout_passes=False
        ),
    )
    def scan_gather(
        idx_hbm, tab_hbm, tail_hbm, out_hbm,
        idx_v, buf_a, buf_b, out_v, sem_a, sem_b, sem_o,
    ):
        tid = lax.axis_index("s") * 2 + lax.axis_index("c")
        bufs = (buf_a, buf_b)
        sems = (sem_a, sem_b)

        def gather_win(buf_v, buf_base, cov_w):
            def _g(g, carry):
                acc = []
                for u in range(4):
                    col = g * 64 + u * 16
                    vv = idx_v[pl.ds(col, 16)]
                    lo = vv - buf_base
                    msk = plsc.bitcast(lo, jnp.uint32) < jnp.uint32(cov_w)
                    pos = lax.iota(jnp.int32, 16) + col
                    vals = [
                        plsc.load_gather(
                            buf_v, [jnp.full((16,), e, jnp.int32), lo], mask=msk
                        )
                        for e in range(8)
                    ]
                    acc.append((pos, msk, vals))
                for pos, msk, vals in acc:
                    for e in range(8):
                        plsc.store_scatter(
                            out_v,
                            [jnp.full((16,), e, jnp.int32), pos],
                            vals[e],
                            mask=msk,
                        )
                return carry

            lax.fori_loop(0, n_g // 4, _g, 0)

        def slab_work(jj, carry):
            slab = tid + 32 * jj

            @pl.when(slab < n_slabs)
            def _process():
                f = slab // 4
                r0 = slab * 8
                pltpu.sync_copy(idx_hbm.at[pl.ds(f * B, B)], idx_v)

                @pl.when(jj > 0)
                def _drain():
                    pltpu.make_async_copy(
                        out_v, out_hbm.at[pl.ds(r0, 8), :], sem_o
                    ).wait()
                def wsrc(w):
                    return tab_hbm.at[pl.ds(r0, 8), pl.ds(w * _VC, _VC)]

                pltpu.async_copy(wsrc(0), buf_a, sem_a)
                pltpu.async_copy(wsrc(1), buf_b, sem_b)
                for w in range(n_win):
                    bsel = w % 2
                    pltpu.make_async_copy(wsrc(w), bufs[bsel], sems[bsel]).wait()
                    gather_win(bufs[bsel], w * _VC, _VC)
                    if w + 2 < n_win:
                        pltpu.async_copy(wsrc(w + 2), bufs[bsel], sems[bsel])
                    elif w + 2 == n_win:
                        pltpu.async_copy(
                            tail_hbm.at[pl.ds(r0, 8), :],
                            bufs[bsel].at[:, pl.ds(0, _TAIL)],
                            sems[bsel],
                        )
                tsel = n_win % 2
                pltpu.make_async_copy(
                    tail_hbm.at[pl.ds(r0, 8), :],
                    bufs[tsel].at[:, pl.ds(0, _TAIL)],
                    sems[tsel],
                ).wait()
                gather_win(bufs[tsel], V - _TAIL, _TAIL)
                pltpu.async_copy(out_v, out_hbm.at[pl.ds(r0, 8), :], sem_o)

            return carry

        lax.fori_loop(0, (n_slabs + 31) // 32, slab_work, 0)
        pltpu.make_async_copy(out_v, out_hbm.at[pl.ds(0, 8), :], sem_o).wait()

    return scan_gather(idx_flat, tab_v, tab_tail)


def _tc_dense_softmax(g_t, W, b, block_b):
    K, B = g_t.shape
    out_dim = W.shape[1]

    def mm_kernel(g_ref, w_ref, b_ref, o_ref):
        logits = lax.dot_general(
            g_ref[...],
            w_ref[...],
            (((0,), (0,)), ((), ())),
            preferred_element_type=jnp.float32,
        )
        logits = logits + b_ref[...]
        m = jnp.max(logits, axis=-1, keepdims=True)
        e = jnp.exp(logits - m)
        o_ref[...] = e / jnp.sum(e, axis=-1, keepdims=True)

    return pl.pallas_call(
        mm_kernel,
        grid=(B // block_b,),
        in_specs=[
            pl.BlockSpec((K, block_b), lambda i: (0, i)),
            pl.BlockSpec((K, out_dim), lambda i: (0, 0)),
            pl.BlockSpec((1, out_dim), lambda i: (0, 0)),
        ],
        out_specs=pl.BlockSpec((block_b, out_dim), lambda i: (i, 0)),
        out_shape=jax.ShapeDtypeStruct((B, out_dim), jnp.float32),
    )(g_t, W, b.reshape(1, out_dim))


def kernel(inputs, batch_size, tables, W, b):
    F, V, E = tables.shape
    B = inputs.shape[0]
    tab_v = tables.transpose(0, 2, 1).reshape(F * E, V)
    idx_flat = inputs.T.reshape(F * B)
    tab_tail = lax.slice(tab_v, (0, V - _TAIL), (F * E, V))

    g_t = _sc_scan_gather(idx_flat, F, tab_v, tab_tail)
    return _tc_dense_softmax(g_t, W, b, block_b=512)


# speedup vs baseline: 1.2733x; 1.0109x over previous
import functools

import jax
import jax.numpy as jnp
from jax import lax
from jax.experimental import pallas as pl
from jax.experimental.pallas import tpu as pltpu
from jax.experimental.pallas import tpu_sc as plsc

_VC = 5504
_TAIL = 1024


def _sc_scan_gather(idx_flat, n_feat, tab_v, tab_tail):
    F = n_feat
    B = idx_flat.shape[0] // F
    K, V = tab_v.shape
    n_slabs = K // 8
    n_win = V // _VC
    tail_lo = n_win * _VC
    assert V - tail_lo <= _TAIL
    n_g = B // 16

    mesh = plsc.VectorSubcoreMesh(core_axis_name="c", subcore_axis_name="s")

    @functools.partial(
        pl.kernel,
        mesh=mesh,
        out_type=jax.ShapeDtypeStruct((K, B), jnp.float32),
        scratch_types=[
            pltpu.VMEM((B,), jnp.int32),
            pltpu.VMEM((8, _VC), jnp.float32),
            pltpu.VMEM((8, _VC), jnp.float32),
            pltpu.VMEM((8, B), jnp.float32),
            pltpu.SemaphoreType.DMA,
            pltpu.SemaphoreType.DMA,
            pltpu.SemaphoreType.DMA,
        ],
        compiler_params=pltpu.CompilerParams(
            use_tc_tiling_on_sc=True, needs_layout_passes=False
        ),
    )
    def scan_gather(
        idx_hbm, tab_hbm, tail_hbm, out_hbm,
        idx_v, buf_a, buf_b, out_v, sem_a, sem_b, sem_o,
    ):
        tid = lax.axis_index("s") * 2 + lax.axis_index("c")
        bufs = (buf_a, buf_b)
        sems = (sem_a, sem_b)

        def gather_win(buf_v, buf_base, cov_w):
            def _g(g, carry):
                acc = []
                for u in range(4):
                    col = g * 64 + u * 16
                    vv = idx_v[pl.ds(col, 16)]
                    lo = vv - buf_base
                    msk = plsc.bitcast(lo, jnp.uint32) < jnp.uint32(cov_w)
                    pos = lax.iota(jnp.int32, 16) + col
                    vals = [
                        plsc.load_gather(
                            buf_v, [jnp.full((16,), e, jnp.int32), lo], mask=msk
                        )
                        for e in range(8)
                    ]
                    acc.append((pos, msk, vals))
                for pos, msk, vals in acc:
                    for e in range(8):
                        plsc.store_scatter(
                            out_v,
                            [jnp.full((16,), e, jnp.int32), pos],
                            vals[e],
                            mask=msk,
                        )
                return carry

            lax.fori_loop(0, n_g // 4, _g, 0)

        def slab_work(jj, carry):
            slab = tid + 32 * jj

            @pl.when(slab < n_slabs)
            def _process():
                f = slab // 4
                r0 = slab * 8
                pltpu.sync_copy(idx_hbm.at[pl.ds(f * B, B)], idx_v)

                @pl.when(jj > 0)
                def _drain():
                    pltpu.make_async_copy(
                        out_v, out_hbm.at[pl.ds(r0, 8), :], sem_o
                    ).wait()
                def wsrc(w):
                    return tab_hbm.at[pl.ds(r0, 8), pl.ds(w * _VC, _VC)]

                for w in range(n_win):
                    bsel = w % 2
                    pltpu.make_async_copy(wsrc(w), bufs[bsel], sems[bsel]).wait()
                    gather_win(bufs[bsel], w * _VC, _VC)
                    if w + 2 < n_win:
                        pltpu.async_copy(wsrc(w + 2), bufs[bsel], sems[bsel])
                    elif w + 2 == n_win:
                        pltpu.async_copy(
                            tail_hbm.at[pl.ds(r0, 8), :],
                            bufs[bsel].at[:, pl.ds(0, _TAIL)],
                            sems[bsel],
                        )
                tsel = n_win % 2
                pltpu.make_async_copy(
                    tail_hbm.at[pl.ds(r0, 8), :],
                    bufs[tsel].at[:, pl.ds(0, _TAIL)],
                    sems[tsel],
                ).wait()
                gather_win(bufs[tsel], V - _TAIL, _TAIL)
                pltpu.async_copy(out_v, out_hbm.at[pl.ds(r0, 8), :], sem_o)

                @pl.when(slab + 32 < n_slabs)
                def _prefetch_next():
                    r0n = r0 + 32 * 8
                    pltpu.async_copy(
                        tab_hbm.at[pl.ds(r0n, 8), pl.ds(0, _VC)], buf_a, sem_a
                    )
                    pltpu.async_copy(
                        tab_hbm.at[pl.ds(r0n, 8), pl.ds(_VC, _VC)], buf_b, sem_b
                    )

            return carry

        pltpu.async_copy(
            tab_hbm.at[pl.ds(tid * 8, 8), pl.ds(0, _VC)], buf_a, sem_a
        )
        pltpu.async_copy(
            tab_hbm.at[pl.ds(tid * 8, 8), pl.ds(_VC, _VC)], buf_b, sem_b
        )
        lax.fori_loop(0, (n_slabs + 31) // 32, slab_work, 0)
        pltpu.make_async_copy(out_v, out_hbm.at[pl.ds(0, 8), :], sem_o).wait()

    return scan_gather(idx_flat, tab_v, tab_tail)


def _tc_dense_softmax(g_t, W, b, block_b):
    K, B = g_t.shape
    out_dim = W.shape[1]

    def mm_kernel(g_ref, w_ref, b_ref, o_ref):
        logits = lax.dot_general(
            g_ref[...],
            w_ref[...],
            (((0,), (0,)), ((), ())),
            preferred_element_type=jnp.float32,
        )
        logits = logits + b_ref[...]
        m = jnp.max(logits, axis=-1, keepdims=True)
        e = jnp.exp(logits - m)
        o_ref[...] = e / jnp.sum(e, axis=-1, keepdims=True)

    return pl.pallas_call(
        mm_kernel,
        grid=(B // block_b,),
        in_specs=[
            pl.BlockSpec((K, block_b), lambda i: (0, i)),
            pl.BlockSpec((K, out_dim), lambda i: (0, 0)),
            pl.BlockSpec((1, out_dim), lambda i: (0, 0)),
        ],
        out_specs=pl.BlockSpec((block_b, out_dim), lambda i: (i, 0)),
        out_shape=jax.ShapeDtypeStruct((B, out_dim), jnp.float32),
    )(g_t, W, b.reshape(1, out_dim))


def kernel(inputs, batch_size, tables, W, b):
    F, V, E = tables.shape
    B = inputs.shape[0]
    tab_v = tables.transpose(0, 2, 1).reshape(F * E, V)
    idx_flat = inputs.T.reshape(F * B)
    tab_tail = lax.slice(tab_v, (0, V - _TAIL), (F * E, V))

    g_t = _sc_scan_gather(idx_flat, F, tab_v, tab_tail)
    return _tc_dense_softmax(g_t, W, b, block_b=512)


# speedup vs baseline: 1.3088x; 1.0279x over previous
import functools

import jax
import jax.numpy as jnp
from jax import lax
from jax.experimental import pallas as pl
from jax.experimental.pallas import tpu as pltpu
from jax.experimental.pallas import tpu_sc as plsc

_VC = 5760
_TAIL = 2176


def _sc_scan_gather(idx_flat, n_feat, tab_v, tab_tail):
    F = n_feat
    B = idx_flat.shape[0] // F
    K, V = tab_v.shape
    n_slabs = K // 8
    n_win = V // _VC
    tail_lo = n_win * _VC
    assert V - tail_lo <= _TAIL
    n_g = B // 16

    mesh = plsc.VectorSubcoreMesh(core_axis_name="c", subcore_axis_name="s")

    @functools.partial(
        pl.kernel,
        mesh=mesh,
        out_type=jax.ShapeDtypeStruct((K, B), jnp.float32),
        scratch_types=[
            pltpu.VMEM((B,), jnp.int32),
            pltpu.VMEM((8, _VC), jnp.float32),
            pltpu.VMEM((8, _VC), jnp.float32),
            pltpu.VMEM((8, B), jnp.float32),
            pltpu.SemaphoreType.DMA,
            pltpu.SemaphoreType.DMA,
            pltpu.SemaphoreType.DMA,
        ],
        compiler_params=pltpu.CompilerParams(
            use_tc_tiling_on_sc=True, needs_layout_passes=False
        ),
    )
    def scan_gather(
        idx_hbm, tab_hbm, tail_hbm, out_hbm,
        idx_v, buf_a, buf_b, out_v, sem_a, sem_b, sem_o,
    ):
        tid = lax.axis_index("s") * 2 + lax.axis_index("c")
        bufs = (buf_a, buf_b)
        sems = (sem_a, sem_b)

        def gather_win(buf_v, buf_base, cov_w):
            def _g(g, carry):
                acc = []
                for u in range(4):
                    col = g * 64 + u * 16
                    vv = idx_v[pl.ds(col, 16)]
                    lo = vv - buf_base
                    msk = plsc.bitcast(lo, jnp.uint32) < jnp.uint32(cov_w)
                    pos = lax.iota(jnp.int32, 16) + col
                    vals = [
                        plsc.load_gather(
                            buf_v, [jnp.full((16,), e, jnp.int32), lo], mask=msk
                        )
                        for e in range(8)
                    ]
                    acc.append((pos, msk, vals))
                for pos, msk, vals in acc:
                    for e in range(8):
                        plsc.store_scatter(
                            out_v,
                            [jnp.full((16,), e, jnp.int32), pos],
                            vals[e],
                            mask=msk,
                        )
                return carry

            lax.fori_loop(0, n_g // 4, _g, 0)

        def slab_work(jj, carry):
            slab = tid + 32 * jj

            @pl.when(slab < n_slabs)
            def _process():
                f = slab // 4
                r0 = slab * 8
                pltpu.sync_copy(idx_hbm.at[pl.ds(f * B, B)], idx_v)

                @pl.when(jj > 0)
                def _drain():
                    pltpu.make_async_copy(
                        out_v, out_hbm.at[pl.ds(r0, 8), :], sem_o
                    ).wait()
                def wsrc(w):
                    return tab_hbm.at[pl.ds(r0, 8), pl.ds(w * _VC, _VC)]

                for w in range(n_win):
                    bsel = w % 2
                    pltpu.make_async_copy(wsrc(w), bufs[bsel], sems[bsel]).wait()
                    gather_win(bufs[bsel], w * _VC, _VC)
                    if w + 2 < n_win:
                        pltpu.async_copy(wsrc(w + 2), bufs[bsel], sems[bsel])
                    elif w + 2 == n_win:
                        pltpu.async_copy(
                            tail_hbm.at[pl.ds(r0, 8), :],
                            bufs[bsel].at[:, pl.ds(0, _TAIL)],
                            sems[bsel],
                        )
                tsel = n_win % 2
                pltpu.make_async_copy(
                    tail_hbm.at[pl.ds(r0, 8), :],
                    bufs[tsel].at[:, pl.ds(0, _TAIL)],
                    sems[tsel],
                ).wait()
                gather_win(bufs[tsel], V - _TAIL, _TAIL)
                pltpu.async_copy(out_v, out_hbm.at[pl.ds(r0, 8), :], sem_o)

                @pl.when(slab + 32 < n_slabs)
                def _prefetch_next():
                    r0n = r0 + 32 * 8
                    pltpu.async_copy(
                        tab_hbm.at[pl.ds(r0n, 8), pl.ds(0, _VC)], buf_a, sem_a
                    )
                    pltpu.async_copy(
                        tab_hbm.at[pl.ds(r0n, 8), pl.ds(_VC, _VC)], buf_b, sem_b
                    )

            return carry

        pltpu.async_copy(
            tab_hbm.at[pl.ds(tid * 8, 8), pl.ds(0, _VC)], buf_a, sem_a
        )
        pltpu.async_copy(
            tab_hbm.at[pl.ds(tid * 8, 8), pl.ds(_VC, _VC)], buf_b, sem_b
        )
        lax.fori_loop(0, (n_slabs + 31) // 32, slab_work, 0)
        pltpu.make_async_copy(out_v, out_hbm.at[pl.ds(0, 8), :], sem_o).wait()

    return scan_gather(idx_flat, tab_v, tab_tail)


def _tc_dense_softmax(g_t, W, b, block_b):
    K, B = g_t.shape
    out_dim = W.shape[1]

    def mm_kernel(g_ref, w_ref, b_ref, o_ref):
        logits = lax.dot_general(
            g_ref[...],
            w_ref[...],
            (((0,), (0,)), ((), ())),
            preferred_element_type=jnp.float32,
        )
        logits = logits + b_ref[...]
        m = jnp.max(logits, axis=-1, keepdims=True)
        e = jnp.exp(logits - m)
        o_ref[...] = e / jnp.sum(e, axis=-1, keepdims=True)

    return pl.pallas_call(
        mm_kernel,
        grid=(B // block_b,),
        in_specs=[
            pl.BlockSpec((K, block_b), lambda i: (0, i)),
            pl.BlockSpec((K, out_dim), lambda i: (0, 0)),
            pl.BlockSpec((1, out_dim), lambda i: (0, 0)),
        ],
        out_specs=pl.BlockSpec((block_b, out_dim), lambda i: (i, 0)),
        out_shape=jax.ShapeDtypeStruct((B, out_dim), jnp.float32),
    )(g_t, W, b.reshape(1, out_dim))


def kernel(inputs, batch_size, tables, W, b):
    F, V, E = tables.shape
    B = inputs.shape[0]
    tab_v = tables.transpose(0, 2, 1).reshape(F * E, V)
    idx_flat = inputs.T.reshape(F * B)
    tab_tail = lax.slice(tab_v, (0, V - _TAIL), (F * E, V))

    g_t = _sc_scan_gather(idx_flat, F, tab_v, tab_tail)
    return _tc_dense_softmax(g_t, W, b, block_b=512)


# speedup vs baseline: 1.3689x; 1.0459x over previous
import functools

import jax
import jax.numpy as jnp
from jax import lax
from jax.experimental import pallas as pl
from jax.experimental.pallas import tpu as pltpu
from jax.experimental.pallas import tpu_sc as plsc

_VC = 5760
_TAIL = 2176


def _sc_scan_gather(idx_flat, n_feat, tab_v, tab_tail):
    F = n_feat
    B = idx_flat.shape[0] // F
    K, V = tab_v.shape
    n_slabs = K // 8
    n_win = V // _VC
    tail_lo = n_win * _VC
    assert V - tail_lo <= _TAIL
    n_g = B // 16

    mesh = plsc.VectorSubcoreMesh(core_axis_name="c", subcore_axis_name="s")

    @functools.partial(
        pl.kernel,
        mesh=mesh,
        out_type=jax.ShapeDtypeStruct((K, B), jnp.float32),
        scratch_types=[
            pltpu.VMEM((B,), jnp.int32),
            pltpu.VMEM((8, _VC), jnp.float32),
            pltpu.VMEM((8, _VC), jnp.float32),
            pltpu.VMEM((8, B), jnp.float32),
            pltpu.SemaphoreType.DMA,
            pltpu.SemaphoreType.DMA,
            pltpu.SemaphoreType.DMA,
        ],
        compiler_params=pltpu.CompilerParams(
            use_tc_tiling_on_sc=True, needs_layout_passes=False
        ),
    )
    def scan_gather(
        idx_hbm, tab_hbm, tail_hbm, out_hbm,
        idx_v, buf_a, buf_b, out_v, sem_a, sem_b, sem_o,
    ):
        tid = lax.axis_index("s") * 2 + lax.axis_index("c")
        bufs = (buf_a, buf_b)
        sems = (sem_a, sem_b)

        def gather_win(buf_v, buf_base, cov_w, col0=0, n_iter=n_g // 4):
            def _g(g, carry):
                acc = []
                for u in range(4):
                    col = col0 + g * 64 + u * 16
                    vv = idx_v[pl.ds(col, 16)]
                    lo = vv - buf_base
                    msk = plsc.bitcast(lo, jnp.uint32) < jnp.uint32(cov_w)
                    pos = lax.iota(jnp.int32, 16) + col
                    vals = [
                        plsc.load_gather(
                            buf_v, [jnp.full((16,), e, jnp.int32), lo], mask=msk
                        )
                        for e in range(8)
                    ]
                    acc.append((pos, msk, vals))
                for pos, msk, vals in acc:
                    for e in range(8):
                        plsc.store_scatter(
                            out_v,
                            [jnp.full((16,), e, jnp.int32), pos],
                            vals[e],
                            mask=msk,
                        )
                return carry

            lax.fori_loop(0, n_iter, _g, 0)

        def slab_work(jj, carry):
            slab = tid + 32 * jj

            @pl.when(slab < n_slabs)
            def _process():
                f = slab // 4
                r0 = slab * 8
                pltpu.sync_copy(idx_hbm.at[pl.ds(f * B, B)], idx_v)

                @pl.when(jj > 0)
                def _drain():
                    pltpu.make_async_copy(
                        out_v, out_hbm.at[pl.ds(r0, 8), :], sem_o
                    ).wait()
                def wsrc(w):
                    return tab_hbm.at[pl.ds(r0, 8), pl.ds(w * _VC, _VC)]

                for w in range(n_win):
                    bsel = w % 2
                    pltpu.make_async_copy(wsrc(w), bufs[bsel], sems[bsel]).wait()
                    gather_win(bufs[bsel], w * _VC, _VC)
                    if w + 2 < n_win:
                        pltpu.async_copy(wsrc(w + 2), bufs[bsel], sems[bsel])
                    elif w + 2 == n_win:
                        pltpu.async_copy(
                            tail_hbm.at[pl.ds(r0, 8), :],
                            bufs[bsel].at[:, pl.ds(0, _TAIL)],
                            sems[bsel],
                        )
                tsel = n_win % 2
                pltpu.make_async_copy(
                    tail_hbm.at[pl.ds(r0, 8), :],
                    bufs[tsel].at[:, pl.ds(0, _TAIL)],
                    sems[tsel],
                ).wait()
                gather_win(bufs[tsel], V - _TAIL, _TAIL)
                pltpu.async_copy(out_v, out_hbm.at[pl.ds(r0, 8), :], sem_o)

                @pl.when((jj < 2) | (tid < 16))
                def _prefetch_next():
                    r0n = jnp.where(jj == 2, (96 + tid // 2) * 8, r0 + 32 * 8)
                    pltpu.async_copy(
                        tab_hbm.at[pl.ds(r0n, 8), pl.ds(0, _VC)], buf_a, sem_a
                    )
                    pltpu.async_copy(
                        tab_hbm.at[pl.ds(r0n, 8), pl.ds(_VC, _VC)], buf_b, sem_b
                    )

            return carry

        pltpu.async_copy(
            tab_hbm.at[pl.ds(tid * 8, 8), pl.ds(0, _VC)], buf_a, sem_a
        )
        pltpu.async_copy(
            tab_hbm.at[pl.ds(tid * 8, 8), pl.ds(_VC, _VC)], buf_b, sem_b
        )
        lax.fori_loop(0, 3, slab_work, 0)

        @pl.when(tid < 16)
        def _round3():
            slab = 96 + tid // 2
            h = tid % 2
            col0 = pl.multiple_of(h * (B // 2), 128)
            f = slab // 4
            r0 = slab * 8
            pltpu.sync_copy(idx_hbm.at[pl.ds(f * B, B)], idx_v)
            pltpu.make_async_copy(out_v, out_hbm.at[pl.ds(r0, 8), :], sem_o).wait()

            def wsrc(w):
                return tab_hbm.at[pl.ds(r0, 8), pl.ds(w * _VC, _VC)]

            for w in range(n_win):
                bsel = w % 2
                pltpu.make_async_copy(wsrc(w), bufs[bsel], sems[bsel]).wait()
                gather_win(bufs[bsel], w * _VC, _VC, col0, B // 2 // 64)
                if w + 2 < n_win:
                    pltpu.async_copy(wsrc(w + 2), bufs[bsel], sems[bsel])
                elif w + 2 == n_win:
                    pltpu.async_copy(
                        tail_hbm.at[pl.ds(r0, 8), :],
                        bufs[bsel].at[:, pl.ds(0, _TAIL)],
                        sems[bsel],
                    )
            tsel = n_win % 2
            pltpu.make_async_copy(
                tail_hbm.at[pl.ds(r0, 8), :],
                bufs[tsel].at[:, pl.ds(0, _TAIL)],
                sems[tsel],
            ).wait()
            gather_win(bufs[tsel], V - _TAIL, _TAIL, col0, B // 2 // 64)
            pltpu.async_copy(
                out_v.at[:, pl.ds(col0, B // 2)],
                out_hbm.at[pl.ds(r0, 8), pl.ds(col0, B // 2)],
                sem_o,
            )

        @pl.when(tid < 16)
        def _drain_half():
            pltpu.make_async_copy(
                out_v.at[:, pl.ds(0, B // 2)],
                out_hbm.at[pl.ds(0, 8), pl.ds(0, B // 2)],
                sem_o,
            ).wait()

        @pl.when(tid >= 16)
        def _drain_full():
            pltpu.make_async_copy(out_v, out_hbm.at[pl.ds(0, 8), :], sem_o).wait()

    return scan_gather(idx_flat, tab_v, tab_tail)


def _tc_dense_softmax(g_t, W, b, block_b):
    K, B = g_t.shape
    out_dim = W.shape[1]

    def mm_kernel(g_ref, w_ref, b_ref, o_ref):
        logits = lax.dot_general(
            g_ref[...],
            w_ref[...],
            (((0,), (0,)), ((), ())),
            preferred_element_type=jnp.float32,
        )
        logits = logits + b_ref[...]
        m = jnp.max(logits, axis=-1, keepdims=True)
        e = jnp.exp(logits - m)
        o_ref[...] = e / jnp.sum(e, axis=-1, keepdims=True)

    return pl.pallas_call(
        mm_kernel,
        grid=(B // block_b,),
        in_specs=[
            pl.BlockSpec((K, block_b), lambda i: (0, i)),
            pl.BlockSpec((K, out_dim), lambda i: (0, 0)),
            pl.BlockSpec((1, out_dim), lambda i: (0, 0)),
        ],
        out_specs=pl.BlockSpec((block_b, out_dim), lambda i: (i, 0)),
        out_shape=jax.ShapeDtypeStruct((B, out_dim), jnp.float32),
    )(g_t, W, b.reshape(1, out_dim))


def kernel(inputs, batch_size, tables, W, b):
    F, V, E = tables.shape
    B = inputs.shape[0]
    tab_v = tables.transpose(0, 2, 1).reshape(F * E, V)
    idx_flat = inputs.T.reshape(F * B)
    tab_tail = lax.slice(tab_v, (0, V - _TAIL), (F * E, V))

    g_t = _sc_scan_gather(idx_flat, F, tab_v, tab_tail)
    return _tc_dense_softmax(g_t, W, b, block_b=512)


# speedup vs baseline: 1.3787x; 1.0072x over previous
import functools

import jax
import jax.numpy as jnp
from jax import lax
from jax.experimental import pallas as pl
from jax.experimental.pallas import tpu as pltpu
from jax.experimental.pallas import tpu_sc as plsc

_VC = 5760
_TAIL = 2176


def _sc_scan_gather(idx_flat, n_feat, tab_v, tab_tail):
    F = n_feat
    B = idx_flat.shape[0] // F
    K, V = tab_v.shape
    n_slabs = K // 8
    n_win = V // _VC
    tail_lo = n_win * _VC
    assert V - tail_lo <= _TAIL
    n_g = B // 16

    mesh = plsc.VectorSubcoreMesh(core_axis_name="c", subcore_axis_name="s")

    @functools.partial(
        pl.kernel,
        mesh=mesh,
        out_type=jax.ShapeDtypeStruct((K, B), jnp.float32),
        scratch_types=[
            pltpu.VMEM((B,), jnp.int32),
            pltpu.VMEM((8, _VC), jnp.float32),
            pltpu.VMEM((8, _VC), jnp.float32),
            pltpu.VMEM((8, B), jnp.float32),
            pltpu.SemaphoreType.DMA,
            pltpu.SemaphoreType.DMA,
            pltpu.SemaphoreType.DMA,
        ],
        compiler_params=pltpu.CompilerParams(
            use_tc_tiling_on_sc=True, needs_layout_passes=False
        ),
    )
    def scan_gather(
        idx_hbm, tab_hbm, tail_hbm, out_hbm,
        idx_v, buf_a, buf_b, out_v, sem_a, sem_b, sem_o,
    ):
        tid = lax.axis_index("s") * 2 + lax.axis_index("c")
        bufs = (buf_a, buf_b)
        sems = (sem_a, sem_b)

        def gather_win(buf_v, buf_base, cov_w, col0=0, n_iter=n_g // 4):
            def _g(g, carry):
                acc = []
                for u in range(4):
                    col = col0 + g * 64 + u * 16
                    vv = idx_v[pl.ds(col, 16)]
                    lo = vv - buf_base
                    msk = plsc.bitcast(lo, jnp.uint32) < jnp.uint32(cov_w)
                    pos = lax.iota(jnp.int32, 16) + col
                    vals = [
                        plsc.load_gather(
                            buf_v, [jnp.full((16,), e, jnp.int32), lo], mask=msk
                        )
                        for e in range(8)
                    ]
                    acc.append((pos, msk, vals))
                for pos, msk, vals in acc:
                    for e in range(8):
                        plsc.store_scatter(
                            out_v,
                            [jnp.full((16,), e, jnp.int32), pos],
                            vals[e],
                            mask=msk,
                        )
                return carry

            lax.fori_loop(0, n_iter, _g, 0)

        def slab_work(jj, carry):
            slab = tid + 32 * jj

            @pl.when(slab < n_slabs)
            def _process():
                f = slab // 4
                r0 = slab * 8
                pltpu.sync_copy(idx_hbm.at[pl.ds(f * B, B)], idx_v)

                @pl.when(jj > 0)
                def _drain():
                    pltpu.make_async_copy(
                        out_v, out_hbm.at[pl.ds(r0, 8), :], sem_o
                    ).wait()
                def wsrc(w):
                    return tab_hbm.at[pl.ds(r0, 8), pl.ds(w * _VC, _VC)]

                for w in range(n_win):
                    bsel = w % 2
                    pltpu.make_async_copy(wsrc(w), bufs[bsel], sems[bsel]).wait()
                    gather_win(bufs[bsel], w * _VC, _VC)
                    if w + 2 < n_win:
                        pltpu.async_copy(wsrc(w + 2), bufs[bsel], sems[bsel])
                    elif w + 2 == n_win:
                        pltpu.async_copy(
                            tail_hbm.at[pl.ds(r0, 8), :],
                            bufs[bsel].at[:, pl.ds(0, _TAIL)],
                            sems[bsel],
                        )
                tsel = n_win % 2
                pltpu.make_async_copy(
                    tail_hbm.at[pl.ds(r0, 8), :],
                    bufs[tsel].at[:, pl.ds(0, _TAIL)],
                    sems[tsel],
                ).wait()
                gather_win(bufs[tsel], V - _TAIL, _TAIL)
                pltpu.async_copy(out_v, out_hbm.at[pl.ds(r0, 8), :], sem_o)

                @pl.when((jj < 2) | (tid < 16))
                def _prefetch_next():
                    r0n = jnp.where(jj == 2, (96 + tid // 2) * 8, r0 + 32 * 8)
                    pltpu.async_copy(
                        tab_hbm.at[pl.ds(r0n, 8), pl.ds(0, _VC)], buf_a, sem_a
                    )
                    pltpu.async_copy(
                        tab_hbm.at[pl.ds(r0n, 8), pl.ds(_VC, _VC)], buf_b, sem_b
                    )

            return carry

        pltpu.async_copy(
            tab_hbm.at[pl.ds(tid * 8, 8), pl.ds(0, _VC)], buf_a, sem_a
        )
        pltpu.async_copy(
            tab_hbm.at[pl.ds(tid * 8, 8), pl.ds(_VC, _VC)], buf_b, sem_b
        )
        lax.fori_loop(0, 3, slab_work, 0)

        @pl.when(tid < 16)
        def _round3():
            slab = 96 + tid // 2
            h = tid % 2
            col0 = pl.multiple_of(h * (B // 2), 128)
            f = slab // 4
            r0 = slab * 8
            pltpu.sync_copy(idx_hbm.at[pl.ds(f * B, B)], idx_v)
            pltpu.make_async_copy(out_v, out_hbm.at[pl.ds(r0, 8), :], sem_o).wait()

            def wsrc(w):
                return tab_hbm.at[pl.ds(r0, 8), pl.ds(w * _VC, _VC)]

            for w in range(n_win):
                bsel = w % 2
                pltpu.make_async_copy(wsrc(w), bufs[bsel], sems[bsel]).wait()
                gather_win(bufs[bsel], w * _VC, _VC, col0, B // 2 // 64)
                if w + 2 < n_win:
                    pltpu.async_copy(wsrc(w + 2), bufs[bsel], sems[bsel])
                elif w + 2 == n_win:
                    pltpu.async_copy(
                        tail_hbm.at[pl.ds(r0, 8), :],
                        bufs[bsel].at[:, pl.ds(0, _TAIL)],
                        sems[bsel],
                    )
            tsel = n_win % 2
            pltpu.make_async_copy(
                tail_hbm.at[pl.ds(r0, 8), :],
                bufs[tsel].at[:, pl.ds(0, _TAIL)],
                sems[tsel],
            ).wait()
            gather_win(bufs[tsel], V - _TAIL, _TAIL, col0, B // 2 // 64)
            pltpu.async_copy(
                out_v.at[:, pl.ds(col0, B // 2)],
                out_hbm.at[pl.ds(r0, 8), pl.ds(col0, B // 2)],
                sem_o,
            )

        @pl.when(tid < 16)
        def _drain_half():
            pltpu.make_async_copy(
                out_v.at[:, pl.ds(0, B // 2)],
                out_hbm.at[pl.ds(0, 8), pl.ds(0, B // 2)],
                sem_o,
            ).wait()

        @pl.when(tid >= 16)
        def _drain_full():
            pltpu.make_async_copy(out_v, out_hbm.at[pl.ds(0, 8), :], sem_o).wait()

    return scan_gather(idx_flat, tab_v, tab_tail)


def _tc_dense_softmax(g_t, W, b, block_b):
    K, B = g_t.shape
    out_dim = W.shape[1]

    def mm_kernel(g_ref, w_ref, b_ref, o_ref):
        logits = lax.dot_general(
            g_ref[...],
            w_ref[...],
            (((0,), (0,)), ((), ())),
            preferred_element_type=jnp.float32,
        )
        logits = logits + b_ref[...]
        m = jnp.max(logits, axis=-1, keepdims=True)
        e = jnp.exp(logits - m)
        o_ref[...] = e / jnp.sum(e, axis=-1, keepdims=True)

    return pl.pallas_call(
        mm_kernel,
        grid=(B // block_b,),
        in_specs=[
            pl.BlockSpec((K, block_b), lambda i: (0, i)),
            pl.BlockSpec((K, out_dim), lambda i: (0, 0)),
            pl.BlockSpec((1, out_dim), lambda i: (0, 0)),
        ],
        out_specs=pl.BlockSpec((block_b, out_dim), lambda i: (i, 0)),
        out_shape=jax.ShapeDtypeStruct((B, out_dim), jnp.float32),
    )(g_t, W, b.reshape(1, out_dim))


def kernel(inputs, batch_size, tables, W, b):
    F, V, E = tables.shape
    B = inputs.shape[0]
    tab_v = tables.transpose(0, 2, 1).reshape(F * E, V)
    idx_flat = inputs.T.reshape(F * B)
    tab_tail = lax.slice(tab_v, (0, V - _TAIL), (F * E, V))

    g_t = _sc_scan_gather(idx_flat, F, tab_v, tab_tail)
    return _tc_dense_softmax(g_t, W, b, block_b=1024)
